# baseline (speedup 1.0000x reference)
import functools

import jax
import jax.numpy as jnp
from jax import lax
from jax.experimental import pallas as pl
from jax.experimental.pallas import tpu as pltpu
from jax.experimental.pallas import tpu_sc as plsc

N = 10000
D = 128
DP = D // 2
E = 320000

NC = 2
NS = 16
NW = NC * NS
B = 80
GB = 5
NG = E // (GB * B)
GPW = NG // NW
NBLK = GPW * GB
NP = 10240
DPT = NP // NS

BD = 128
GBD = 10
NGD = E // (GBD * BD)

_mesh = plsc.VectorSubcoreMesh(core_axis_name="c", subcore_axis_name="s")


@functools.partial(
    pl.kernel,
    out_type=jax.ShapeDtypeStruct((NC, NP), jnp.float32),
    mesh=_mesh,
    scratch_types=[
        pltpu.VMEM((GBD, BD), jnp.int32),
        pltpu.VMEM((GBD, BD), jnp.float32),
        pltpu.VMEM_SHARED((NP,), jnp.float32),
        pltpu.SemaphoreType.DMA,
    ],
)
def _deg_kernel(col_hbm, w_hbm, zero_hbm, out_hbm, cidx, wv, acc, sem):
    c = lax.axis_index("c")
    s = lax.axis_index("s")
    wid = c * NS + s
    rslc = pl.ds(s * DPT, DPT)
    pltpu.sync_copy(zero_hbm.at[rslc], acc.at[rslc])
    plsc.subcore_barrier()
    g0 = wid * NGD // NW
    g1 = (wid + 1) * NGD // NW

    def gbody(g, carry):
        pltpu.sync_copy(col_hbm.at[g], cidx)
        pltpu.sync_copy(w_hbm.at[g], wv)
        for b in range(GBD):
            pltpu.async_copy(wv.at[b], acc.at[cidx.at[b]], sem, add=True)
        for b in range(GBD):
            pltpu.make_async_copy(wv.at[b], acc.at[cidx.at[b]], sem).wait()
        return carry

    lax.fori_loop(g0, g1, gbody, 0)
    plsc.subcore_barrier()
    pltpu.sync_copy(acc.at[rslc], out_hbm.at[c, rslc])


@functools.partial(
    pl.kernel,
    out_type=jax.ShapeDtypeStruct((NC, NP, D), jnp.float32),
    mesh=_mesh,
    scratch_types=[
        pltpu.VMEM((2, GB, B), jnp.int32),
        pltpu.VMEM((2, GB, B), jnp.int32),
        pltpu.VMEM((2, GB, B), jnp.float32),
        pltpu.VMEM((2, B, DP), jnp.float32),
        pltpu.VMEM((2, B, D), jnp.float32),
        pltpu.VMEM_SHARED((NP, D), jnp.float32),
        [pltpu.SemaphoreType.DMA] * 2,
        [pltpu.SemaphoreType.DMA] * 2,
        pltpu.SemaphoreType.DMA,
    ],
    compiler_params=pltpu.CompilerParams(use_tc_tiling_on_sc=False),
)
def _agg_kernel(h_hbm, row_hbm, col_hbm, w_hbm, zero_hbm, out_hbm,
                ridx, cidx, wv, rin, rout, acc, gsem, ssem, stsem):
    c = lax.axis_index("c")
    s = lax.axis_index("s")
    wid = c * NS + s
    rslc = pl.ds(s * DPT, DPT)
    pltpu.sync_copy(zero_hbm.at[rslc], acc.at[rslc])
    plsc.subcore_barrier()
    g0 = wid * GPW

    def stage(g, slot):
        pltpu.async_copy(row_hbm.at[g], ridx.at[slot], stsem)
        pltpu.async_copy(col_hbm.at[g], cidx.at[slot], stsem)
        pltpu.async_copy(w_hbm.at[g], wv.at[slot], stsem)

    def stage_wait(g, slot):
        pltpu.make_async_copy(row_hbm.at[g], ridx.at[slot], stsem).wait()
        pltpu.make_async_copy(col_hbm.at[g], cidx.at[slot], stsem).wait()
        pltpu.make_async_copy(w_hbm.at[g], wv.at[slot], stsem).wait()

    pltpu.sync_copy(row_hbm.at[g0], ridx.at[0])
    pltpu.sync_copy(col_hbm.at[g0], cidx.at[0])
    pltpu.sync_copy(w_hbm.at[g0], wv.at[0])
    pltpu.async_copy(h_hbm.at[ridx.at[0, 0]], rin.at[0], gsem[0])

    def step(t, par):
        oth = 1 - par
        g = g0 + t // GB
        b = t % GB
        slot = (t // GB) % 2

        @pl.when(t + 1 < NBLK)
        def _():
            t1 = t + 1
            g2 = g0 + t1 // GB
            b2 = t1 % GB
            slot2 = (t1 // GB) % 2

            @pl.when(b2 == 0)
            def _():
                stage_wait(g2, slot2)

            pltpu.async_copy(h_hbm.at[ridx.at[slot2, b2]], rin.at[oth],
                             gsem[oth])

        pltpu.make_async_copy(h_hbm.at[ridx.at[slot, b]], rin.at[par],
                              gsem[par]).wait()

        @pl.when(t >= 2)
        def _():
            pltpu.make_async_copy(rout.at[par], acc.at[cidx.at[0, 0]],
                                  ssem[par]).wait()

        @pl.when(jnp.logical_and(b == 1, g + 1 < g0 + GPW))
        def _():
            stage(g + 1, 1 - slot)

        def scale(gg, inner):
            w16 = wv[slot, b, pl.ds(gg * 16, 16)]
            for l in range(16):
                wsc = w16[l]
                e = gg * 16 + l
                for k in range(DP // 16):
                    v = rin[par, e, pl.ds(k * 16, 16)]
                    vi = lax.bitcast_convert_type(v, jnp.int32)
                    va = lax.bitcast_convert_type(
                        lax.shift_left(vi, 16), jnp.float32)
                    vb = lax.bitcast_convert_type(
                        jnp.bitwise_and(vi, jnp.int32(-65536)), jnp.float32)
                    rout[par, e, pl.ds(k * 32, 16)] = va * wsc
                    rout[par, e, pl.ds(k * 32 + 16, 16)] = vb * wsc
            return inner

        lax.fori_loop(0, B // 16, scale, 0)
        pltpu.async_copy(rout.at[par], acc.at[cidx.at[slot, b]], ssem[par],
                         add=True)

    def body(t, carry):
        for par in range(2):
            @pl.when(t % 2 == par)
            def _(par=par):
                step(t, par)
        return carry

    lax.fori_loop(0, NBLK, body, 0)
    pltpu.make_async_copy(rout.at[0], acc.at[cidx.at[0, 0]], ssem[0]).wait()
    pltpu.make_async_copy(rout.at[1], acc.at[cidx.at[0, 0]], ssem[1]).wait()
    plsc.subcore_barrier()
    pltpu.sync_copy(acc.at[rslc], out_hbm.at[c, rslc])


RB = 1000
GRID = N // RB


def _mm1_body(degp_ref, x_ref, w_ref, dinv_ref, h_ref):
    deg = degp_ref[:, 0:1] + degp_ref[:, 1:2] + 1.0
    dinv = jnp.where(deg > 0, lax.rsqrt(deg), 0.0)
    dinv_ref[...] = dinv
    h = jnp.dot(x_ref[...], w_ref[...], preferred_element_type=jnp.float32)
    h_ref[...] = dinv * h


def _mid_body(q_ref, hhat_ref, dinv_ref, b_ref, w_ref, out_ref):
    dinv = dinv_ref[...]
    z = dinv * (q_ref[0] + q_ref[1] + hhat_ref[...]) + b_ref[...]
    z = jnp.maximum(z, 0.0)
    h = jnp.dot(z, w_ref[...], preferred_element_type=jnp.float32)
    out_ref[...] = dinv * h


def _fin_body(q_ref, hhat_ref, dinv_ref, b_ref, w_ref, bp_ref, out_ref):
    dinv = dinv_ref[...]
    z = dinv * (q_ref[0] + q_ref[1] + hhat_ref[...]) + b_ref[...]
    z = jnp.maximum(z, 0.0)
    out_ref[...] = (
        jnp.dot(z, w_ref[...], preferred_element_type=jnp.float32) + bp_ref[...]
    )


_rows2 = pl.BlockSpec((RB, D), lambda i: (i, 0))
_rows3 = pl.BlockSpec((NC, RB, D), lambda i: (0, i, 0))
_rows1 = pl.BlockSpec((RB, 1), lambda i: (i, 0))
_full2 = pl.BlockSpec((D, D), lambda i: (0, 0))
_fullb = pl.BlockSpec((D,), lambda i: (0,))

_mm1 = pl.pallas_call(
    _mm1_body,
    grid=(GRID,),
    in_specs=[pl.BlockSpec((RB, NC), lambda i: (i, 0)), _rows2, _full2],
    out_specs=[_rows1, _rows2],
    out_shape=[
        jax.ShapeDtypeStruct((N, 1), jnp.float32),
        jax.ShapeDtypeStruct((N, D), jnp.float32),
    ],
)

_mid = pl.pallas_call(
    _mid_body,
    grid=(GRID,),
    in_specs=[_rows3, _rows2, _rows1, _fullb, _full2],
    out_specs=_rows2,
    out_shape=jax.ShapeDtypeStruct((N, D), jnp.float32),
)

_fin = pl.pallas_call(
    _fin_body,
    grid=(GRID,),
    in_specs=[_rows3, _rows2, _rows1, _fullb, _full2, _fullb],
    out_specs=_rows2,
    out_shape=jax.ShapeDtypeStruct((N, D), jnp.float32),
)


def _pack_bf16(h):
    hh = h.reshape(N, D // 32, 2, 16).transpose(0, 1, 3, 2)
    hb = hh.astype(jnp.bfloat16).reshape(N, DP, 2)
    return jax.lax.bitcast_convert_type(hb, jnp.float32)


def kernel(x, edge_index, edge_attr, W1, b1, W2, b2, Wp, bp):
    row = edge_index[0].reshape(NG, GB, B)
    col = edge_index[1].reshape(NG, GB, B)
    w = edge_attr.reshape(NG, GB, B)
    col_d = edge_index[1].reshape(NGD, GBD, BD)
    w_d = edge_attr.reshape(NGD, GBD, BD)
    zero1 = jnp.zeros((NP,), jnp.float32)
    zero2 = jnp.zeros((NP, D), jnp.float32)

    degp = _deg_kernel(col_d, w_d, zero1)
    dinv, hhat1 = _mm1(degp[:, :N].T, x, W1)
    q1 = _agg_kernel(_pack_bf16(hhat1), row, col, w, zero2)
    hhat2 = _mid(q1, hhat1, dinv, b1, W2)
    q2 = _agg_kernel(_pack_bf16(hhat2), row, col, w, zero2)
    return _fin(q2, hhat2, dinv, b2, Wp, bp)


# speedup vs baseline: 1.8069x; 1.8069x over previous
import functools

import jax
import jax.numpy as jnp
from jax import lax
from jax.experimental import pallas as pl
from jax.experimental.pallas import tpu as pltpu
from jax.experimental.pallas import tpu_sc as plsc

N = 10000
D = 128
E = 320000

NC = 2
NS = 16
NW = NC * NS
B = 80
GB = 5
NG = E // (GB * B)
GPW = NG // NW
NBLK = GPW * GB
NP = 10240
DPT = NP // NS

BD = 128
GBD = 10
NGD = E // (GBD * BD)

_mesh = plsc.VectorSubcoreMesh(core_axis_name="c", subcore_axis_name="s")


@functools.partial(
    pl.kernel,
    out_type=jax.ShapeDtypeStruct((NC, NP), jnp.float32),
    mesh=_mesh,
    scratch_types=[
        pltpu.VMEM((GBD, BD), jnp.int32),
        pltpu.VMEM((GBD, BD), jnp.float32),
        pltpu.VMEM_SHARED((NP,), jnp.float32),
        pltpu.SemaphoreType.DMA,
    ],
)
def _deg_kernel(col_hbm, w_hbm, zero_hbm, out_hbm, cidx, wv, acc, sem):
    c = lax.axis_index("c")
    s = lax.axis_index("s")
    wid = c * NS + s
    rslc = pl.ds(s * DPT, DPT)
    pltpu.sync_copy(zero_hbm.at[rslc], acc.at[rslc])
    plsc.subcore_barrier()
    g0 = wid * NGD // NW
    g1 = (wid + 1) * NGD // NW

    def gbody(g, carry):
        pltpu.sync_copy(col_hbm.at[g], cidx)
        pltpu.sync_copy(w_hbm.at[g], wv)
        for b in range(GBD):
            pltpu.async_copy(wv.at[b], acc.at[cidx.at[b]], sem, add=True)
        for b in range(GBD):
            pltpu.make_async_copy(wv.at[b], acc.at[cidx.at[b]], sem).wait()
        return carry

    lax.fori_loop(g0, g1, gbody, 0)
    plsc.subcore_barrier()
    pltpu.sync_copy(acc.at[rslc], out_hbm.at[c, rslc])


@functools.partial(
    pl.kernel,
    out_type=jax.ShapeDtypeStruct((NC, NP, D), jnp.float32),
    mesh=_mesh,
    scratch_types=[
        pltpu.VMEM((2, GB, B), jnp.int32),
        pltpu.VMEM((2, GB, B), jnp.int32),
        pltpu.VMEM((2, GB, B), jnp.float32),
        pltpu.VMEM((3, B, D), jnp.float32),
        pltpu.VMEM_SHARED((NP, D), jnp.float32),
        [pltpu.SemaphoreType.DMA] * 3,
        [pltpu.SemaphoreType.DMA] * 3,
        pltpu.SemaphoreType.DMA,
    ],
)
def _agg_kernel(h_hbm, row_hbm, col_hbm, w_hbm, zero_hbm, out_hbm,
                ridx, cidx, wv, rows, acc, gsem, ssem, stsem):
    c = lax.axis_index("c")
    s = lax.axis_index("s")
    wid = c * NS + s
    rslc = pl.ds(s * DPT, DPT)
    pltpu.sync_copy(zero_hbm.at[rslc], acc.at[rslc])
    plsc.subcore_barrier()
    g0 = wid * GPW

    def stage(g, slot):
        pltpu.async_copy(row_hbm.at[g], ridx.at[slot], stsem)
        pltpu.async_copy(col_hbm.at[g], cidx.at[slot], stsem)
        pltpu.async_copy(w_hbm.at[g], wv.at[slot], stsem)

    def stage_wait(g, slot):
        pltpu.make_async_copy(row_hbm.at[g], ridx.at[slot], stsem).wait()
        pltpu.make_async_copy(col_hbm.at[g], cidx.at[slot], stsem).wait()
        pltpu.make_async_copy(w_hbm.at[g], wv.at[slot], stsem).wait()

    pltpu.sync_copy(row_hbm.at[g0], ridx.at[0])
    pltpu.sync_copy(col_hbm.at[g0], cidx.at[0])
    pltpu.sync_copy(w_hbm.at[g0], wv.at[0])
    pltpu.async_copy(h_hbm.at[ridx.at[0, 0]], rows.at[0], gsem[0])
    pltpu.async_copy(h_hbm.at[ridx.at[0, 1]], rows.at[1], gsem[1])

    def step(t, par):
        nxt = (par + 1) % 3
        prv = (par + 2) % 3
        g = g0 + t // GB
        b = t % GB
        slot = (t // GB) % 2

        @pl.when(t >= 1)
        def _():
            pltpu.make_async_copy(rows.at[prv], acc.at[cidx.at[0, 0]],
                                  ssem[prv]).wait()

        @pl.when(jnp.logical_and(b == 1, g + 1 < g0 + GPW))
        def _():
            stage(g + 1, 1 - slot)

        @pl.when(t + 2 < NBLK)
        def _():
            t2 = t + 2
            g2 = g0 + t2 // GB
            b2 = t2 % GB
            slot2 = (t2 // GB) % 2

            @pl.when(b2 == 0)
            def _():
                stage_wait(g2, slot2)

            pltpu.async_copy(h_hbm.at[ridx.at[slot2, b2]], rows.at[prv],
                             gsem[prv])

        pltpu.make_async_copy(h_hbm.at[ridx.at[slot, b]], rows.at[par],
                              gsem[par]).wait()

        def scale(gg, inner):
            w16 = wv[slot, b, pl.ds(gg * 16, 16)]
            for l in range(16):
                wsc = w16[l]
                e = gg * 16 + l
                for k in range(D // 16):
                    sl = pl.ds(k * 16, 16)
                    rows[par, e, sl] = rows[par, e, sl] * wsc
            return inner

        lax.fori_loop(0, B // 16, scale, 0)
        pltpu.async_copy(rows.at[par], acc.at[cidx.at[slot, b]], ssem[par],
                         add=True)

    def body(t, carry):
        for par in range(3):
            @pl.when(t % 3 == par)
            def _(par=par):
                step(t, par)
        return carry

    lax.fori_loop(0, NBLK, body, 0)
    pltpu.make_async_copy(rows.at[1], acc.at[cidx.at[0, 0]], ssem[1]).wait()
    plsc.subcore_barrier()
    pltpu.sync_copy(acc.at[rslc], out_hbm.at[c, rslc])


RB = 1000
GRID = N // RB


def _mm1_body(degp_ref, x_ref, w_ref, dinv_ref, h_ref):
    deg = degp_ref[:, 0:1] + degp_ref[:, 1:2] + 1.0
    dinv = jnp.where(deg > 0, lax.rsqrt(deg), 0.0)
    dinv_ref[...] = dinv
    h = jnp.dot(x_ref[...], w_ref[...], preferred_element_type=jnp.float32)
    h_ref[...] = dinv * h


def _mid_body(q_ref, hhat_ref, dinv_ref, b_ref, w_ref, out_ref):
    dinv = dinv_ref[...]
    z = dinv * (q_ref[0] + q_ref[1] + hhat_ref[...]) + b_ref[...]
    z = jnp.maximum(z, 0.0)
    h = jnp.dot(z, w_ref[...], preferred_element_type=jnp.float32)
    out_ref[...] = dinv * h


def _fin_body(q_ref, hhat_ref, dinv_ref, b_ref, w_ref, bp_ref, out_ref):
    dinv = dinv_ref[...]
    z = dinv * (q_ref[0] + q_ref[1] + hhat_ref[...]) + b_ref[...]
    z = jnp.maximum(z, 0.0)
    out_ref[...] = (
        jnp.dot(z, w_ref[...], preferred_element_type=jnp.float32) + bp_ref[...]
    )


_rows2 = pl.BlockSpec((RB, D), lambda i: (i, 0))
_rows3 = pl.BlockSpec((NC, RB, D), lambda i: (0, i, 0))
_rows1 = pl.BlockSpec((RB, 1), lambda i: (i, 0))
_full2 = pl.BlockSpec((D, D), lambda i: (0, 0))
_fullb = pl.BlockSpec((D,), lambda i: (0,))

_mm1 = pl.pallas_call(
    _mm1_body,
    grid=(GRID,),
    in_specs=[pl.BlockSpec((RB, NC), lambda i: (i, 0)), _rows2, _full2],
    out_specs=[_rows1, _rows2],
    out_shape=[
        jax.ShapeDtypeStruct((N, 1), jnp.float32),
        jax.ShapeDtypeStruct((N, D), jnp.float32),
    ],
)

_mid = pl.pallas_call(
    _mid_body,
    grid=(GRID,),
    in_specs=[_rows3, _rows2, _rows1, _fullb, _full2],
    out_specs=_rows2,
    out_shape=jax.ShapeDtypeStruct((N, D), jnp.float32),
)

_fin = pl.pallas_call(
    _fin_body,
    grid=(GRID,),
    in_specs=[_rows3, _rows2, _rows1, _fullb, _full2, _fullb],
    out_specs=_rows2,
    out_shape=jax.ShapeDtypeStruct((N, D), jnp.float32),
)


def kernel(x, edge_index, edge_attr, W1, b1, W2, b2, Wp, bp):
    row = edge_index[0].reshape(NG, GB, B)
    col = edge_index[1].reshape(NG, GB, B)
    w = edge_attr.reshape(NG, GB, B)
    col_d = edge_index[1].reshape(NGD, GBD, BD)
    w_d = edge_attr.reshape(NGD, GBD, BD)
    zero1 = jnp.zeros((NP,), jnp.float32)
    zero2 = jnp.zeros((NP, D), jnp.float32)

    degp = _deg_kernel(col_d, w_d, zero1)
    dinv, hhat1 = _mm1(degp[:, :N].T, x, W1)
    q1 = _agg_kernel(hhat1, row, col, w, zero2)
    hhat2 = _mid(q1, hhat1, dinv, b1, W2)
    q2 = _agg_kernel(hhat2, row, col, w, zero2)
    return _fin(q2, hhat2, dinv, b2, Wp, bp)
